# baseline (speedup 1.0000x reference)
import functools

import jax
import jax.numpy as jnp
from jax import lax
from jax.experimental import pallas as pl
from jax.experimental.pallas import tpu as pltpu
from jax.experimental.pallas import tpu_sc as plsc

_NC = 2
_NS = 16
_NW = _NC * _NS
_LANES = 16
_NB = 2
_C = 40


@jax.jit
def _u_mul_e(h, src, affine):
    E, D = affine.shape
    N = h.shape[0]
    assert E % (_NW * _C) == 0
    ew = E // _NW
    n_chunks = ew // _C
    n_groups = n_chunks // _NB
    assert n_chunks % _NB == 0 and n_groups >= 2
    vregs_per_row = D // _LANES

    mesh = plsc.VectorSubcoreMesh(core_axis_name="c", subcore_axis_name="s")

    buf_types = [pltpu.VMEM((_C, D), jnp.float32) for _ in range(3 * _NB)]
    sem_types = [pltpu.SemaphoreType.DMA for _ in range(3 * _NB)]

    @functools.partial(
        pl.kernel,
        mesh=mesh,
        out_type=jax.ShapeDtypeStruct((E, D), jnp.float32),
        scratch_types=[pltpu.VMEM((ew,), jnp.int32)] + buf_types + sem_types,
    )
    def run(h_hbm, src_hbm, aff_hbm, out_hbm, idx_v, *rest):
        rows = rest[0:_NB]
        aff = rest[_NB:2 * _NB]
        outb = rest[2 * _NB:3 * _NB]
        gsem = rest[3 * _NB:4 * _NB]
        asem = rest[4 * _NB:5 * _NB]
        ssem = rest[5 * _NB:6 * _NB]

        wid = lax.axis_index("s") * _NC + lax.axis_index("c")
        base_w = wid * ew
        pltpu.sync_copy(src_hbm.at[pl.ds(base_w, ew)], idx_v)

        def issue_loads(i, b):
            pltpu.async_copy(
                h_hbm.at[idx_v.at[pl.ds(i * _C, _C)]], rows[b], gsem[b])
            pltpu.async_copy(
                aff_hbm.at[pl.ds(base_w + i * _C, _C)], aff[b], asem[b])

        def wait_loads(b):
            pltpu.make_async_copy(
                h_hbm.at[pl.ds(0, _C)], rows[b], gsem[b]).wait()

            pltpu.make_async_copy(
                aff_hbm.at[pl.ds(0, _C)], aff[b], asem[b]).wait()

        def issue_store(i, b):
            pltpu.async_copy(
                outb[b], out_hbm.at[pl.ds(base_w + i * _C, _C)], ssem[b])

        def wait_store(b):
            pltpu.make_async_copy(
                outb[b], out_hbm.at[pl.ds(0, _C)], ssem[b]).wait()

        def compute(b):
            def row(r, rc):
                for v in range(vregs_per_row):
                    sl = pl.ds(v * _LANES, _LANES)
                    outb[b][r, sl] = rows[b][r, sl] * aff[b][r, sl]
                return rc

            lax.fori_loop(0, _C, row, 0)

        for b in range(_NB - 1):
            issue_loads(b, b)

        for b in range(_NB):
            issue_loads(b + _NB - 1, (b + _NB - 1) % _NB)
            wait_loads(b)
            compute(b)
            issue_store(b, b)

        def group(g, carry):
            i0 = g * _NB
            for b in range(_NB):
                issue_loads(i0 + b + _NB - 1, (b + _NB - 1) % _NB)
                wait_loads(b)
                wait_store(b)
                compute(b)
                issue_store(i0 + b, b)
            return carry

        lax.fori_loop(1, n_groups - 1, group, 0)

        i0 = (n_groups - 1) * _NB
        issue_loads(i0 + _NB - 1, (_NB - 1) % _NB)
        for b in range(_NB):
            wait_loads(b)
            wait_store(b)
            compute(b)
            issue_store(i0 + b, b)

        for b in range(_NB):
            wait_store(b)

    return run(h, src, affine)


def kernel(h, affine, edge_index):
    return _u_mul_e(h, edge_index[0], affine)


# speedup vs baseline: 1.3730x; 1.3730x over previous
import functools

import jax
import jax.numpy as jnp
from jax import lax
from jax.experimental import pallas as pl
from jax.experimental.pallas import tpu as pltpu
from jax.experimental.pallas import tpu_sc as plsc

_NC = 2
_NS = 16
_NW = _NC * _NS
_LANES = 16
_NB = 3
_NI = 6
_C = 40


@jax.jit
def _u_mul_e(h, src, affine):
    E, D = affine.shape
    N = h.shape[0]
    assert E % (_NW * _C) == 0
    ew = E // _NW
    n_chunks = ew // _C
    n_steady = (n_chunks - _NI) // _NI
    n_tail = n_chunks - _NI * (1 + n_steady)
    vregs_per_row = D // _LANES

    mesh = plsc.VectorSubcoreMesh(core_axis_name="c", subcore_axis_name="s")

    scratch = (
        [pltpu.VMEM_SHARED((N, D), jnp.float32)]
        + [pltpu.VMEM((_C, D), jnp.float32) for _ in range(3 * _NB)]
        + [pltpu.VMEM((_C,), jnp.int32) for _ in range(_NI)]
        + [pltpu.SemaphoreType.DMA for _ in range(3 * _NB + _NI)]
    )

    @functools.partial(
        pl.kernel,
        mesh=mesh,
        out_type=jax.ShapeDtypeStruct((E, D), jnp.float32),
        scratch_types=scratch,
    )
    def run(h_hbm, src_hbm, aff_hbm, out_hbm, h_sh, *rest):
        rows = rest[0:_NB]
        aff = rest[_NB:2 * _NB]
        outb = rest[2 * _NB:3 * _NB]
        idxb = rest[3 * _NB:3 * _NB + _NI]
        gsem = rest[3 * _NB + _NI:4 * _NB + _NI]
        asem = rest[4 * _NB + _NI:5 * _NB + _NI]
        ssem = rest[5 * _NB + _NI:6 * _NB + _NI]
        isem = rest[6 * _NB + _NI:6 * _NB + 2 * _NI]

        sid = lax.axis_index("s")
        wid = sid * _NC + lax.axis_index("c")
        base_w = wid * ew

        @pl.when(sid == 0)
        def _():
            pltpu.sync_copy(h_hbm, h_sh)
        plsc.subcore_barrier()

        def issue_idx(i, s):
            pltpu.async_copy(
                src_hbm.at[pl.ds(base_w + i * _C, _C)], idxb[s], isem[s])

        def wait_idx(s):
            pltpu.make_async_copy(
                src_hbm.at[pl.ds(0, _C)], idxb[s], isem[s]).wait()

        def issue_loads(i, s, b):
            pltpu.async_copy(h_sh.at[idxb[s]], rows[b], gsem[b])
            pltpu.async_copy(
                aff_hbm.at[pl.ds(base_w + i * _C, _C)], aff[b], asem[b])

        def wait_loads(b):
            pltpu.make_async_copy(
                h_hbm.at[pl.ds(0, _C)], rows[b], gsem[b]).wait()
            pltpu.make_async_copy(
                aff_hbm.at[pl.ds(0, _C)], aff[b], asem[b]).wait()

        def issue_store(i, b):
            pltpu.async_copy(
                outb[b], out_hbm.at[pl.ds(base_w + i * _C, _C)], ssem[b])

        def wait_store(b):
            pltpu.make_async_copy(
                outb[b], out_hbm.at[pl.ds(0, _C)], ssem[b]).wait()

        def compute(b):
            def row(r, rc):
                for v in range(vregs_per_row):
                    sl = pl.ds(v * _LANES, _LANES)
                    outb[b][r, sl] = rows[b][r, sl] * aff[b][r, sl]
                return rc

            lax.fori_loop(0, _C, row, 0)

        def body(i, k, store_wait, la2, la4):
            b = k % _NB
            if la4:
                issue_idx(i + 4, (k + 4) % _NI)
            if la2:
                wait_idx((k + 2) % _NI)
                issue_loads(i + 2, (k + 2) % _NI, (k + 2) % _NB)
            wait_loads(b)
            if store_wait:
                wait_store(b)
            compute(b)
            issue_store(i, b)

        for j in range(4):
            issue_idx(j, j)
        for j in range(2):
            wait_idx(j)
            issue_loads(j, j, j)

        for k in range(_NI):
            body(k, k, store_wait=(k >= _NB), la2=True, la4=True)

        def group(g, carry):
            i0 = g * _NI
            for k in range(_NI):
                body(i0 + k, k, store_wait=True, la2=True, la4=True)
            return carry

        lax.fori_loop(1, 1 + n_steady, group, 0)

        i0 = (1 + n_steady) * _NI
        for t in range(n_tail):
            i = i0 + t
            body(i, t, store_wait=True,
                 la2=(i + 2 < n_chunks), la4=(i + 4 < n_chunks))

        for i in range(n_chunks - _NB, n_chunks):
            wait_store(i % _NB)

    return run(h, src, affine)


def kernel(h, affine, edge_index):
    return _u_mul_e(h, edge_index[0], affine)
